# baseline (speedup 1.0000x reference)
import jax
import jax.numpy as jnp
from jax import lax
from jax.experimental import pallas as pl
from jax.experimental.pallas import tpu as pltpu
from jax.experimental.pallas import tpu_sc as plsc

NBX = 512
NBY = 512
NB = NBX * NBY
NN = 1_000_000
NP = 900_000
NM = 800_000
H = 0.7071067811865476
MINR = 1.0 / 2.5
MAXR = 2.5

NC = 2
NS = 16
NW = NC * NS
SL = NB // NS

CS = 2048
NCH_S = -(-NP // CS)
ITER_S = -(-NCH_S // NW)

CG = 2048
NCH_G = -(-NM // CG)
ITER_G = -(-NCH_G // NW)

_mesh = plsc.VectorSubcoreMesh(core_axis_name="c", subcore_axis_name="s")

_scatter_scratch = (
    [pltpu.VMEM((CS,), jnp.float32)] * 10
    + [
        pltpu.VMEM((9 * CS,), jnp.int32),
        pltpu.VMEM((9 * CS,), jnp.float32),
        pltpu.VMEM((9 * CS,), jnp.int32),
        pltpu.VMEM((9 * CS,), jnp.float32),
        pltpu.VMEM_SHARED((NB,), jnp.float32),
        pltpu.SemaphoreType.DMA,
        pltpu.SemaphoreType.DMA,
    ]
)


@jax.jit
def _scatter_call(pos, nsx, nsy, w):
    return pl.kernel(
        _scatter_body,
        mesh=_mesh,
        out_type=jax.ShapeDtypeStruct((NC, NB), jnp.float32),
        scratch_types=_scatter_scratch,
    )(pos, nsx, nsy, w)


def _scatter_body(pos, nsx, nsy, w, out, *scratch):
    bufs = [scratch[0:5], scratch[5:10]]
    sbufs = [scratch[10:12], scratch[12:14]]
    map_sh, ldsem, ssem = scratch[14:17]
    val_v = sbufs[0][1]
    cid = lax.axis_index("c")
    sid = lax.axis_index("s")
    wid = cid * NS + sid
    iota = lax.iota(jnp.int32, 16)
    zeros16 = jnp.zeros((16,), jnp.float32)

    def zbody(i, carry):
        val_v[pl.ds(i * 16, 16)] = zeros16
        return carry

    lax.fori_loop(0, SL // 16, zbody, 0)
    pltpu.sync_copy(val_v.at[pl.ds(0, SL)], map_sh.at[pl.ds(sid * SL, SL)])
    plsc.subcore_barrier()

    def chunk_base(k):
        raw = wid + k * NW
        chunk = jnp.minimum(raw, NCH_S - 1)
        return jnp.minimum(chunk * CS, NP - CS), raw * CS

    def issue_loads(k, p):
        base, _ = chunk_base(k)
        bu = bufs[p]
        return [
            pltpu.async_copy(pos.at[pl.ds(base, CS)], bu[0], ldsem),
            pltpu.async_copy(pos.at[pl.ds(NN + base, CS)], bu[1], ldsem),
            pltpu.async_copy(nsx.at[pl.ds(base, CS)], bu[2], ldsem),
            pltpu.async_copy(nsy.at[pl.ds(base, CS)], bu[3], ldsem),
            pltpu.async_copy(w.at[pl.ds(base, CS)], bu[4], ldsem),
        ]

    pend_loads = issue_loads(0, 0)
    pend_scat = [None, None]
    for k in range(ITER_S):
        p = k & 1
        px_v, py_v, sx_v, sy_v, w_v = bufs[p]
        idx_v, val_v = sbufs[p]
        base, rawb = chunk_base(k)
        for d_ in pend_loads:
            d_.wait()
        if k + 1 < ITER_S:
            pend_loads = issue_loads(k + 1, (k + 1) & 1)
        if pend_scat[p] is not None:
            pend_scat[p].wait()

        def gbody(g, c2):
            o = g * 16
            px = px_v[pl.ds(o, 16)]
            py = py_v[pl.ds(o, 16)]
            sx = sx_v[pl.ds(o, 16)]
            sy = sy_v[pl.ds(o, 16)]
            ww = w_v[pl.ds(o, 16)]
            gv = base + o + iota
            m = (gv >= rawb) & (gv < NP)
            cx = px + 0.5 * sx
            cy = py + 0.5 * sy
            xmn = cx - H
            xmx = cx + H
            ymn = cy - H
            ymx = cy + H
            den = ww / ((xmx - xmn) * (ymx - ymn))
            den = jnp.where(m, den, 0.0)
            bxi = jnp.maximum(xmn.astype(jnp.int32), 0)
            byi = jnp.maximum(ymn.astype(jnp.int32), 0)
            bxf = bxi.astype(jnp.float32)
            byf = byi.astype(jnp.float32)
            kb = bxi * NBY + byi
            for d in range(3):
                lo = bxf + float(d)
                ox = jnp.minimum(xmx, lo + 1.0) - jnp.maximum(xmn, lo)
                ox = jnp.maximum(ox, 0.0) * den
                for e in range(3):
                    lo2 = byf + float(e)
                    oy = jnp.minimum(ymx, lo2 + 1.0) - jnp.maximum(ymn, lo2)
                    oy = jnp.maximum(oy, 0.0)
                    s = (3 * d + e) * CS + o
                    val_v[pl.ds(s, 16)] = ox * oy
                    idx_v[pl.ds(s, 16)] = kb + (d * NBY + e)
            return c2

        lax.fori_loop(0, CS // 16, gbody, 0)
        pend_scat[p] = pltpu.async_copy(val_v, map_sh.at[idx_v], ssem,
                                        add=True)

    for pd in pend_scat:
        if pd is not None:
            pd.wait()
    plsc.subcore_barrier()
    pltpu.sync_copy(map_sh.at[pl.ds(sid * SL, SL)],
                    out.at[cid, pl.ds(sid * SL, SL)])


_gather_scratch = (
    [
        pltpu.VMEM((SL + 16,), jnp.float32),
        pltpu.VMEM((SL,), jnp.float32),
        pltpu.VMEM((SL,), jnp.int32),
    ]
    + [
        pltpu.VMEM((CG,), jnp.float32),
        pltpu.VMEM((CG,), jnp.float32),
        pltpu.VMEM((CG,), jnp.float32),
        pltpu.VMEM((CG,), jnp.float32),
        pltpu.VMEM((2 * CG,), jnp.int32),
        pltpu.VMEM((4 * CG,), jnp.float32),
        pltpu.VMEM((2 * CG,), jnp.int32),
        pltpu.VMEM((CG,), jnp.float32),
    ] * 2
    + [
        pltpu.VMEM_SHARED((NB,), jnp.int32),
        pltpu.SemaphoreType.DMA,
        pltpu.SemaphoreType.DMA,
    ]
)


@jax.jit
def _gather_call(maps, pos, nsx, nsy):
    return pl.kernel(
        _gather_body,
        mesh=_mesh,
        out_type=jax.ShapeDtypeStruct((NM,), jnp.float32),
        scratch_types=_gather_scratch,
    )(maps, pos, nsx, nsy)


def _gather_body(maps, pos, nsx, nsy, out, *scratch):
    m0_v, m1_v, pk_v = scratch[0:3]
    bufs = [scratch[3:11], scratch[11:19]]
    map_sh, ldsem, gsem = scratch[19:22]
    cid = lax.axis_index("c")
    sid = lax.axis_index("s")
    wid = cid * NS + sid

    c0 = pltpu.async_copy(maps.at[0, pl.ds(sid * SL, SL)],
                          m0_v.at[pl.ds(0, SL)], ldsem)
    c1 = pltpu.async_copy(maps.at[1, pl.ds(sid * SL, SL)], m1_v, ldsem)
    c0.wait()
    c1.wait()

    def ubody(i, carry):
        o = i * 16
        u = (m0_v[pl.ds(o, 16)] + m1_v[pl.ds(o, 16)]) * 0.0625
        u = jnp.minimum(jnp.maximum(u, MINR), MAXR)
        m0_v[pl.ds(o, 16)] = u
        return carry

    lax.fori_loop(0, SL // 16, ubody, 0)

    rbias = jnp.full((16,), 0x7FFF, jnp.int32)
    one16 = jnp.full((16,), 1, jnp.int32)
    himask16 = jnp.full((16,), -65536, jnp.int32)

    def pbody(i, carry):
        o = i * 16
        ia = lax.bitcast_convert_type(m0_v[pl.ds(o, 16)], jnp.int32)
        ib = lax.bitcast_convert_type(m0_v[pl.ds(o + 1, 16)], jnp.int32)
        ia = ia + rbias + ((ia >> 16) & one16)
        ib = ib + rbias + ((ib >> 16) & one16)
        pk_v[pl.ds(o, 16)] = ((ia >> 16) & jnp.full((16,), 0xFFFF, jnp.int32)
                              ) | (ib & himask16)
        return carry

    lax.fori_loop(0, SL // 16, pbody, 0)
    pltpu.sync_copy(pk_v, map_sh.at[pl.ds(sid * SL, SL)])
    plsc.subcore_barrier()

    def chunk_base(k):
        raw = wid + k * NW
        chunk = jnp.minimum(raw, NCH_G - 1)
        return jnp.minimum(chunk * CG, NM - CG)

    def issue_loads(k, p):
        base = chunk_base(k)
        bu = bufs[p]
        return [
            pltpu.async_copy(pos.at[pl.ds(base, CG)], bu[0], ldsem),
            pltpu.async_copy(pos.at[pl.ds(NN + base, CG)], bu[1], ldsem),
            pltpu.async_copy(nsx.at[pl.ds(base, CG)], bu[2], ldsem),
            pltpu.async_copy(nsy.at[pl.ds(base, CG)], bu[3], ldsem),
        ]

    def compute_idx(k, p):
        px_v, py_v, sx_v, sy_v, idx_v, wgt_v, _, _ = bufs[p]

        def gbody(g, c2):
            o = g * 16
            xmn = px_v[pl.ds(o, 16)]
            ymn = py_v[pl.ds(o, 16)]
            xmx = xmn + sx_v[pl.ds(o, 16)]
            ymx = ymn + sy_v[pl.ds(o, 16)]
            bxi = xmn.astype(jnp.int32)
            byi = ymn.astype(jnp.int32)
            bxf = bxi.astype(jnp.float32)
            byf = byi.astype(jnp.float32)
            kb = bxi * NBY + byi
            for d in range(2):
                lo = bxf + float(d)
                ox = jnp.minimum(xmx, lo + 1.0) - jnp.maximum(xmn, lo)
                ox = jnp.maximum(ox, 0.0)
                idx_v[pl.ds(d * CG + o, 16)] = kb + d * NBY
                for e in range(2):
                    lo2 = byf + float(e)
                    oy = jnp.minimum(ymx, lo2 + 1.0) - jnp.maximum(ymn, lo2)
                    oy = jnp.maximum(oy, 0.0)
                    s = (2 * d + e) * CG + o
                    wgt_v[pl.ds(s, 16)] = ox * oy
            return c2

        lax.fori_loop(0, CG // 16, gbody, 0)

    himask = jnp.full((16,), -65536, jnp.int32)

    def drain(k, p):
        _, _, _, _, _, wgt_v, gat_v, area_v = bufs[p]

        def abody(g, c2):
            o = g * 16
            a = None
            for d in range(2):
                gpk = gat_v[pl.ds(d * CG + o, 16)]
                ulo = lax.bitcast_convert_type(gpk << 16, jnp.float32)
                uhi = lax.bitcast_convert_type(gpk & himask, jnp.float32)
                t = (wgt_v[pl.ds(2 * d * CG + o, 16)] * ulo
                     + wgt_v[pl.ds((2 * d + 1) * CG + o, 16)] * uhi)
                a = t if a is None else a + t
            area_v[pl.ds(o, 16)] = a
            return c2

        lax.fori_loop(0, CG // 16, abody, 0)
        pltpu.sync_copy(area_v, out.at[pl.ds(chunk_base(k), CG)])

    pend_loads = issue_loads(0, 0)
    pend_gather = None
    for k in range(ITER_G):
        p = k & 1
        for d_ in pend_loads:
            d_.wait()
        if k + 1 < ITER_G:
            pend_loads = issue_loads(k + 1, (k + 1) & 1)
        compute_idx(k, p)
        idx_v, gat_v = bufs[p][4], bufs[p][6]
        this_gather = pltpu.async_copy(map_sh.at[idx_v], gat_v, gsem)
        if pend_gather is not None:
            pend_gather.wait()
            drain(k - 1, p ^ 1)
        pend_gather = this_gather
    pend_gather.wait()
    drain(ITER_G - 1, (ITER_G - 1) & 1)


def kernel(pos, node_size_x, node_size_y, pin_weights):
    maps = _scatter_call(pos, node_size_x, node_size_y, pin_weights)
    return _gather_call(maps, pos, node_size_x, node_size_y)


# speedup vs baseline: 1.0677x; 1.0677x over previous
import jax
import jax.numpy as jnp
from jax import lax
from jax.experimental import pallas as pl
from jax.experimental.pallas import tpu as pltpu
from jax.experimental.pallas import tpu_sc as plsc

NBX = 512
NBY = 512
NB = NBX * NBY
NN = 1_000_000
NP = 900_000
NM = 800_000
H = 0.7071067811865476
MINR = 1.0 / 2.5
MAXR = 2.5

NC = 2
NS = 16
NW = NC * NS
SL = NB // NS

CS = 2048
NCH_S = -(-NP // CS)
ITER_S = -(-NCH_S // NW)

CG = 2048
NCH_G = -(-NM // CG)
ITER_G = -(-NCH_G // NW)

_mesh = plsc.VectorSubcoreMesh(core_axis_name="c", subcore_axis_name="s")

_scatter_scratch = (
    [pltpu.VMEM((CS,), jnp.float32)] * 10
    + [
        pltpu.VMEM((9 * CS,), jnp.int32),
        pltpu.VMEM((9 * CS,), jnp.float32),
        pltpu.VMEM((9 * CS,), jnp.int32),
        pltpu.VMEM((9 * CS,), jnp.float32),
        pltpu.VMEM_SHARED((NB,), jnp.float32),
        pltpu.SemaphoreType.DMA,
        pltpu.SemaphoreType.DMA,
    ]
)


@jax.jit
def _scatter_call(pos, nsx, nsy, w):
    return pl.kernel(
        _scatter_body,
        mesh=_mesh,
        out_type=jax.ShapeDtypeStruct((NC, NB), jnp.float32),
        scratch_types=_scatter_scratch,
    )(pos, nsx, nsy, w)


def _scatter_body(pos, nsx, nsy, w, out, *scratch):
    bufs = [scratch[0:5], scratch[5:10]]
    sbufs = [scratch[10:12], scratch[12:14]]
    map_sh, ldsem, ssem = scratch[14:17]
    val_v = sbufs[0][1]
    cid = lax.axis_index("c")
    sid = lax.axis_index("s")
    wid = cid * NS + sid
    iota = lax.iota(jnp.int32, 16)
    zeros16 = jnp.zeros((16,), jnp.float32)

    def zbody(i, carry):
        val_v[pl.ds(i * 16, 16)] = zeros16
        return carry

    lax.fori_loop(0, SL // 16, zbody, 0)
    pltpu.sync_copy(val_v.at[pl.ds(0, SL)], map_sh.at[pl.ds(sid * SL, SL)])
    plsc.subcore_barrier()

    def chunk_base(k):
        raw = wid + k * NW
        chunk = jnp.minimum(raw, NCH_S - 1)
        return jnp.minimum(chunk * CS, NP - CS), raw * CS

    def issue_loads(k, p):
        base, _ = chunk_base(k)
        bu = bufs[p]
        return [
            pltpu.async_copy(pos.at[pl.ds(base, CS)], bu[0], ldsem),
            pltpu.async_copy(pos.at[pl.ds(NN + base, CS)], bu[1], ldsem),
            pltpu.async_copy(nsx.at[pl.ds(base, CS)], bu[2], ldsem),
            pltpu.async_copy(nsy.at[pl.ds(base, CS)], bu[3], ldsem),
            pltpu.async_copy(w.at[pl.ds(base, CS)], bu[4], ldsem),
        ]

    pend_loads = issue_loads(0, 0)
    pend_scat = [None, None]
    for k in range(ITER_S):
        p = k & 1
        px_v, py_v, sx_v, sy_v, w_v = bufs[p]
        idx_v, val_v = sbufs[p]
        base, rawb = chunk_base(k)
        for d_ in pend_loads:
            d_.wait()
        if k + 1 < ITER_S:
            pend_loads = issue_loads(k + 1, (k + 1) & 1)
        if pend_scat[p] is not None:
            pend_scat[p].wait()

        def gbody(g, c2):
            o = g * 16
            px = px_v[pl.ds(o, 16)]
            py = py_v[pl.ds(o, 16)]
            sx = sx_v[pl.ds(o, 16)]
            sy = sy_v[pl.ds(o, 16)]
            ww = w_v[pl.ds(o, 16)]
            gv = base + o + iota
            m = (gv >= rawb) & (gv < NP)
            cx = px + 0.5 * sx
            cy = py + 0.5 * sy
            xmn = cx - H
            xmx = cx + H
            ymn = cy - H
            ymx = cy + H
            den = ww / ((xmx - xmn) * (ymx - ymn))
            den = jnp.where(m, den, 0.0)
            bxi = jnp.maximum(xmn.astype(jnp.int32), 0)
            byi = jnp.maximum(ymn.astype(jnp.int32), 0)
            bxf = bxi.astype(jnp.float32)
            byf = byi.astype(jnp.float32)
            kb = bxi * NBY + byi
            for d in range(3):
                lo = bxf + float(d)
                ox = jnp.minimum(xmx, lo + 1.0) - jnp.maximum(xmn, lo)
                ox = jnp.maximum(ox, 0.0) * den
                for e in range(3):
                    lo2 = byf + float(e)
                    oy = jnp.minimum(ymx, lo2 + 1.0) - jnp.maximum(ymn, lo2)
                    oy = jnp.maximum(oy, 0.0)
                    s = (3 * d + e) * CS + o
                    val_v[pl.ds(s, 16)] = ox * oy
                    idx_v[pl.ds(s, 16)] = kb + (d * NBY + e)
            return c2

        lax.fori_loop(0, CS // 16, gbody, 0)
        pend_scat[p] = pltpu.async_copy(val_v, map_sh.at[idx_v], ssem,
                                        add=True)

    for pd in pend_scat:
        if pd is not None:
            pd.wait()
    plsc.subcore_barrier()
    pltpu.sync_copy(map_sh.at[pl.ds(sid * SL, SL)],
                    out.at[cid, pl.ds(sid * SL, SL)])


_gather_scratch = (
    [
        pltpu.VMEM((SL,), jnp.float32),
        pltpu.VMEM((SL,), jnp.float32),
    ]
    + [
        pltpu.VMEM((CG,), jnp.float32),
        pltpu.VMEM((CG,), jnp.float32),
        pltpu.VMEM((CG,), jnp.float32),
        pltpu.VMEM((CG,), jnp.float32),
        pltpu.VMEM((4 * CG,), jnp.int32),
        pltpu.VMEM((4 * CG,), jnp.float32),
        pltpu.VMEM((4 * CG,), jnp.float32),
        pltpu.VMEM((CG,), jnp.float32),
    ] * 2
    + [
        pltpu.VMEM_SHARED((NB,), jnp.float32),
        pltpu.SemaphoreType.DMA,
        pltpu.SemaphoreType.DMA,
    ]
)


@jax.jit
def _gather_call(maps, pos, nsx, nsy):
    return pl.kernel(
        _gather_body,
        mesh=_mesh,
        out_type=jax.ShapeDtypeStruct((NM,), jnp.float32),
        scratch_types=_gather_scratch,
    )(maps, pos, nsx, nsy)


def _gather_body(maps, pos, nsx, nsy, out, *scratch):
    m0_v, m1_v = scratch[0:2]
    bufs = [scratch[2:10], scratch[10:18]]
    map_sh, ldsem, gsem = scratch[18:21]
    cid = lax.axis_index("c")
    sid = lax.axis_index("s")
    wid = cid * NS + sid

    c0 = pltpu.async_copy(maps.at[0, pl.ds(sid * SL, SL)], m0_v, ldsem)
    c1 = pltpu.async_copy(maps.at[1, pl.ds(sid * SL, SL)], m1_v, ldsem)
    c0.wait()
    c1.wait()

    def ubody(i, carry):
        o = i * 16
        u = (m0_v[pl.ds(o, 16)] + m1_v[pl.ds(o, 16)]) * 0.0625
        u = jnp.minimum(jnp.maximum(u, MINR), MAXR)
        m0_v[pl.ds(o, 16)] = u
        return carry

    lax.fori_loop(0, SL // 16, ubody, 0)
    pltpu.sync_copy(m0_v, map_sh.at[pl.ds(sid * SL, SL)])
    plsc.subcore_barrier()

    def chunk_base(k):
        raw = wid + k * NW
        chunk = jnp.minimum(raw, NCH_G - 1)
        return jnp.minimum(chunk * CG, NM - CG)

    def issue_loads(k, p):
        base = chunk_base(k)
        bu = bufs[p]
        return [
            pltpu.async_copy(pos.at[pl.ds(base, CG)], bu[0], ldsem),
            pltpu.async_copy(pos.at[pl.ds(NN + base, CG)], bu[1], ldsem),
            pltpu.async_copy(nsx.at[pl.ds(base, CG)], bu[2], ldsem),
            pltpu.async_copy(nsy.at[pl.ds(base, CG)], bu[3], ldsem),
        ]

    def compute_idx(k, p):
        px_v, py_v, sx_v, sy_v, idx_v, wgt_v, _, _ = bufs[p]

        def gbody(g, c2):
            o = g * 16
            xmn = px_v[pl.ds(o, 16)]
            ymn = py_v[pl.ds(o, 16)]
            xmx = xmn + sx_v[pl.ds(o, 16)]
            ymx = ymn + sy_v[pl.ds(o, 16)]
            bxi = xmn.astype(jnp.int32)
            byi = ymn.astype(jnp.int32)
            bxf = bxi.astype(jnp.float32)
            byf = byi.astype(jnp.float32)
            kb = bxi * NBY + byi
            for d in range(2):
                lo = bxf + float(d)
                ox = jnp.minimum(xmx, lo + 1.0) - jnp.maximum(xmn, lo)
                ox = jnp.maximum(ox, 0.0)
                for e in range(2):
                    lo2 = byf + float(e)
                    oy = jnp.minimum(ymx, lo2 + 1.0) - jnp.maximum(ymn, lo2)
                    oy = jnp.maximum(oy, 0.0)
                    s = (2 * d + e) * CG + o
                    wgt_v[pl.ds(s, 16)] = ox * oy
                    idx_v[pl.ds(s, 16)] = kb + (d * NBY + e)
            return c2

        lax.fori_loop(0, CG // 16, gbody, 0)

    def drain(k, p):
        _, _, _, _, _, wgt_v, gat_v, area_v = bufs[p]

        def abody(g, c2):
            o = g * 16
            a = wgt_v[pl.ds(o, 16)] * gat_v[pl.ds(o, 16)]
            for q in range(1, 4):
                s = q * CG + o
                a = a + wgt_v[pl.ds(s, 16)] * gat_v[pl.ds(s, 16)]
            area_v[pl.ds(o, 16)] = a
            return c2

        lax.fori_loop(0, CG // 16, abody, 0)
        pltpu.sync_copy(area_v, out.at[pl.ds(chunk_base(k), CG)])

    pend_loads = issue_loads(0, 0)
    pend_gather = None
    for k in range(ITER_G):
        p = k & 1
        for d_ in pend_loads:
            d_.wait()
        if k + 1 < ITER_G:
            pend_loads = issue_loads(k + 1, (k + 1) & 1)
        compute_idx(k, p)
        idx_v, gat_v = bufs[p][4], bufs[p][6]
        this_gather = pltpu.async_copy(map_sh.at[idx_v], gat_v, gsem)
        if pend_gather is not None:
            pend_gather.wait()
            drain(k - 1, p ^ 1)
        pend_gather = this_gather
    pend_gather.wait()
    drain(ITER_G - 1, (ITER_G - 1) & 1)


def kernel(pos, node_size_x, node_size_y, pin_weights):
    maps = _scatter_call(pos, node_size_x, node_size_y, pin_weights)
    return _gather_call(maps, pos, node_size_x, node_size_y)
